# Initial kernel scaffold; baseline (speedup 1.0000x reference)
import jax
import jax.numpy as jnp
from jax.experimental import pallas as pl


def kernel(x, code_book):
    raise NotImplementedError("write your pallas kernel here")


# speedup vs baseline: 2.0861x; 2.0861x over previous
import jax
import jax.numpy as jnp
from jax.experimental import pallas as pl

_DIM = 64
_KS = 1024
_DEPTH = 4


def _vq_kernel(x_ref, cb_ref, out_ref):
    r = x_ref[0]
    cb = cb_ref[...]
    s = jnp.zeros_like(r)
    row_iota = jax.lax.broadcasted_iota(jnp.int32, (_KS, r.shape[1]), 0)
    for _ in range(_DEPTH):
        ip = jax.lax.dot_general(
            cb, r, (((1,), (0,)), ((), ())),
            preferred_element_type=jnp.float32)
        code = jnp.argmax(ip, axis=0)
        u = jnp.max(ip, axis=0)
        onehot = jnp.where(row_iota == code[None, :], 1.0, 0.0)
        g = jax.lax.dot_general(
            cb, onehot, (((0,), (0,)), ((), ())),
            preferred_element_type=jnp.float32,
            precision=jax.lax.Precision.HIGHEST)
        comp = g * u[None, :]
        s = s + comp
        r = r - comp
    out_ref[0] = s


def kernel(x, code_book):
    B, C, H, W = x.shape
    xf = x.reshape(B, C, H * W)
    out = pl.pallas_call(
        _vq_kernel,
        grid=(B,),
        in_specs=[
            pl.BlockSpec((1, C, H * W), lambda b: (b, 0, 0)),
            pl.BlockSpec((_KS, _DIM), lambda b: (0, 0)),
        ],
        out_specs=pl.BlockSpec((1, C, H * W), lambda b: (b, 0, 0)),
        out_shape=jax.ShapeDtypeStruct((B, C, H * W), x.dtype),
    )(xf, code_book)
    return out.reshape(B, C, H, W)


# speedup vs baseline: 2.6053x; 1.2489x over previous
import jax
import jax.numpy as jnp
from jax.experimental import pallas as pl

_DIM = 64
_KS = 1024
_DEPTH = 4
_NGRP = _KS // 128


def _vq_kernel(x_ref, cb_ref, cbt_ref, out_ref):
    r = x_ref[0]
    cb = cb_ref[...]
    nt = r.shape[1]
    s = jnp.zeros_like(r)
    for _ in range(_DEPTH):
        ip = jax.lax.dot_general(
            cb, r, (((1,), (0,)), ((), ())),
            preferred_element_type=jnp.float32)
        code = jnp.argmax(ip, axis=0)
        u = jnp.max(ip, axis=0)
        lane = jnp.broadcast_to((code & 127)[None, :], (_DIM, nt))
        grp = code >> 7
        g = jnp.zeros((_DIM, nt), jnp.float32)
        for q in range(_NGRP):
            cand = jnp.take_along_axis(
                cbt_ref[:, q * 128:(q + 1) * 128], lane, axis=1)
            g = jnp.where((grp == q)[None, :], cand, g)
        comp = g * u[None, :]
        s = s + comp
        r = r - comp
    out_ref[0] = s


def kernel(x, code_book):
    B, C, H, W = x.shape
    xf = x.reshape(B, C, H * W)
    out = pl.pallas_call(
        _vq_kernel,
        grid=(B,),
        in_specs=[
            pl.BlockSpec((1, C, H * W), lambda b: (b, 0, 0)),
            pl.BlockSpec((_KS, _DIM), lambda b: (0, 0)),
            pl.BlockSpec((_DIM, _KS), lambda b: (0, 0)),
        ],
        out_specs=pl.BlockSpec((1, C, H * W), lambda b: (b, 0, 0)),
        out_shape=jax.ShapeDtypeStruct((B, C, H * W), x.dtype),
    )(xf, code_book, code_book.T)
    return out.reshape(B, C, H, W)
